# Initial kernel scaffold; baseline (speedup 1.0000x reference)
import jax
import jax.numpy as jnp
from jax.experimental import pallas as pl


def kernel(h, ind_id, W, att_w):
    raise NotImplementedError("write your pallas kernel here")


# speedup vs baseline: 1.5609x; 1.5609x over previous
import jax
import jax.numpy as jnp
from jax.experimental import pallas as pl

BLK = 256


def _wh_kernel(h_ref, w_ref, wh_ref):
    wh_ref[...] = jax.lax.dot_general(
        h_ref[...], w_ref[...], (((1,), (1,)), ((), ())),
        preferred_element_type=jnp.float32)


def _attn_kernel(whb_ref, wh_ref, indr_ref, indc_ref, ai_ref, aj_ref, ac_ref,
                 out_ref):
    wh_blk = whb_ref[...]
    fi = jnp.dot(wh_blk, ai_ref[...], preferred_element_type=jnp.float32)
    fjt = jax.lax.dot_general(
        aj_ref[...], wh_ref[...], (((1,), (1,)), ((), ())),
        preferred_element_type=jnp.float32)
    e = fi + fjt + ac_ref[...]
    e = jnp.where(e >= 0, e, 0.1 * e)
    mask = indr_ref[...] == indc_ref[...]
    e = jnp.where(mask, e, -1e9)
    m = jnp.max(e, axis=1, keepdims=True)
    p = jnp.exp(e - m)
    s = jnp.sum(p, axis=1, keepdims=True)
    alpha = p / s
    out = jnp.dot(alpha, wh_ref[...], preferred_element_type=jnp.float32)
    cnt = jnp.sum(mask.astype(jnp.int32), axis=1, keepdims=True)
    out_ref[...] = jnp.where(cnt > 1, out, 0.0)


def kernel(h, ind_id, W, att_w):
    n, hid = h.shape
    wh = pl.pallas_call(
        _wh_kernel,
        out_shape=jax.ShapeDtypeStruct((n, hid), jnp.float32),
    )(h, W)

    a = att_w[0]
    ai = a[:hid].reshape(hid, 1)
    aj = a[hid:2 * hid].reshape(1, hid)
    ac = a[2 * hid].reshape(1, 1)
    indr = ind_id.reshape(n, 1)
    indc = ind_id.reshape(1, n)

    out = pl.pallas_call(
        _attn_kernel,
        grid=(n // BLK,),
        in_specs=[
            pl.BlockSpec((BLK, hid), lambda r: (r, 0)),
            pl.BlockSpec((n, hid), lambda r: (0, 0)),
            pl.BlockSpec((BLK, 1), lambda r: (r, 0)),
            pl.BlockSpec((1, n), lambda r: (0, 0)),
            pl.BlockSpec((hid, 1), lambda r: (0, 0)),
            pl.BlockSpec((1, hid), lambda r: (0, 0)),
            pl.BlockSpec((1, 1), lambda r: (0, 0)),
        ],
        out_specs=pl.BlockSpec((BLK, hid), lambda r: (r, 0)),
        out_shape=jax.ShapeDtypeStruct((n, hid), jnp.float32),
    )(wh, wh, indr, indc, ai, aj, ac)
    return out


# speedup vs baseline: 1.6495x; 1.0568x over previous
import jax
import jax.numpy as jnp
from jax.experimental import pallas as pl
from jax.experimental.pallas import tpu as pltpu

BLK = 256


def _wh_kernel(h_ref, w_ref, ai_ref, aj_ref, wh_ref, fi_ref, fjt_ref):
    wh = jax.lax.dot_general(
        h_ref[...], w_ref[...], (((1,), (1,)), ((), ())),
        preferred_element_type=jnp.float32)
    fi_ref[...] = jnp.dot(wh, ai_ref[...], preferred_element_type=jnp.float32)
    fjt_ref[...] = jax.lax.dot_general(
        aj_ref[...], wh, (((1,), (1,)), ((), ())),
        preferred_element_type=jnp.float32)
    wh_ref[...] = wh


def _attn_kernel(blkf_ref, blkl_ref, ac_ref, wh_ref, fi_ref, fjt_ref,
                 indr_ref, indc_ref, out_ref, acc_ref, m_ref, s_ref, cnt_ref):
    r = pl.program_id(0)
    nb = pl.num_programs(0)
    acc_ref[...] = jnp.zeros_like(acc_ref)
    m_ref[...] = jnp.full_like(m_ref, -1e30)
    s_ref[...] = jnp.zeros_like(s_ref)
    cnt_ref[...] = jnp.zeros_like(cnt_ref)
    r_first = blkf_ref[r]
    r_last = blkl_ref[r]
    fi = fi_ref[...]
    ids_r = indr_ref[...]
    ac = ac_ref[0, 0]

    def body(c, carry):
        @pl.when((blkf_ref[c] <= r_last) & (blkl_ref[c] >= r_first))
        def _process():
            e = fi + fjt_ref[:, pl.ds(c * BLK, BLK)] + ac
            e = jnp.where(e >= 0, e, 0.1 * e)
            mask = ids_r == indc_ref[:, pl.ds(c * BLK, BLK)]
            e = jnp.where(mask, e, -1e9)
            m_old = m_ref[...]
            m_new = jnp.maximum(m_old, jnp.max(e, axis=1, keepdims=True))
            p = jnp.where(mask, jnp.exp(e - m_new), 0.0)
            scale = jnp.exp(m_old - m_new)
            whc = wh_ref[pl.ds(c * BLK, BLK), :]
            acc_ref[...] = acc_ref[...] * scale + jax.lax.dot_general(
                p, whc, (((1,), (0,)), ((), ())),
                preferred_element_type=jnp.float32)
            s_ref[...] = s_ref[...] * scale + jnp.sum(p, axis=1, keepdims=True)
            m_ref[...] = m_new
            cnt_ref[...] = cnt_ref[...] + jnp.sum(
                mask.astype(jnp.int32), axis=1, keepdims=True)
        return carry

    jax.lax.fori_loop(0, nb, body, 0)
    out_ref[...] = jnp.where(cnt_ref[...] > 1,
                             acc_ref[...] / s_ref[...], 0.0)


def kernel(h, ind_id, W, att_w):
    n, hid = h.shape
    a = att_w[0]
    ai = a[:hid].reshape(hid, 1)
    aj = a[hid:2 * hid].reshape(1, hid)
    ac = a[2 * hid].reshape(1, 1)

    wh, fi, fjt = pl.pallas_call(
        _wh_kernel,
        out_shape=(
            jax.ShapeDtypeStruct((n, hid), jnp.float32),
            jax.ShapeDtypeStruct((n, 1), jnp.float32),
            jax.ShapeDtypeStruct((1, n), jnp.float32),
        ),
    )(h, W, ai, aj)

    indr = ind_id.reshape(n, 1)
    indc = ind_id.reshape(1, n)
    blk_first = ind_id[0::BLK]
    blk_last = ind_id[BLK - 1::BLK]

    out = pl.pallas_call(
        _attn_kernel,
        grid=(n // BLK,),
        in_specs=[
            pl.BlockSpec(memory_space=pltpu.SMEM),
            pl.BlockSpec(memory_space=pltpu.SMEM),
            pl.BlockSpec(memory_space=pltpu.SMEM),
            pl.BlockSpec((n, hid), lambda r: (0, 0)),
            pl.BlockSpec((BLK, 1), lambda r: (r, 0)),
            pl.BlockSpec((1, n), lambda r: (0, 0)),
            pl.BlockSpec((BLK, 1), lambda r: (r, 0)),
            pl.BlockSpec((1, n), lambda r: (0, 0)),
        ],
        out_specs=pl.BlockSpec((BLK, hid), lambda r: (r, 0)),
        out_shape=jax.ShapeDtypeStruct((n, hid), jnp.float32),
        scratch_shapes=[
            pltpu.VMEM((BLK, hid), jnp.float32),
            pltpu.VMEM((BLK, 1), jnp.float32),
            pltpu.VMEM((BLK, 1), jnp.float32),
            pltpu.VMEM((BLK, 1), jnp.int32),
        ],
    )(blk_first, blk_last, ac, wh, fi, fjt, indr, indc)
    return out
